# baseline (speedup 1.0000x reference)
import jax
import jax.numpy as jnp
from jax import lax
from jax.experimental import pallas as pl
from jax.experimental.pallas import tpu as pltpu
from jax.experimental.pallas import tpu_sc as plsc

_NC, _NS = 2, 16
_NW = _NC * _NS
_BT = 128
_NG = 8
_NP = 2
_G = 6


def _emb_body(x_hbm, w_hbm, out_hbm, idx_v, idxt_v, rows_g, pt, gsem, ssem):
    wid = lax.axis_index("s") * _NC + lax.axis_index("c")
    ns = idxt_v.shape[0]
    d = w_hbm.shape[1]

    pltpu.sync_copy(x_hbm.at[pl.ds(wid * _BT, _BT)], idx_v)
    iota = jax.lax.iota(jnp.int32, 16)

    def sloop(s_, carry):
        scol = jnp.full((16,), 0, jnp.int32) + s_
        for m in range(_BT // 16):
            vec = plsc.load_gather(idx_v, [iota + m * 16, scol])
            plsc.store_scatter(idxt_v, [scol, iota + m * 16], vec)
        return carry

    lax.fori_loop(0, ns, sloop, 0)

    def fire_gather(s_, slot):
        pltpu.async_copy(w_hbm.at[idxt_v.at[s_]], rows_g.at[slot], gsem)

    def drain_gather(slot):
        pltpu.make_async_copy(
            w_hbm.at[idxt_v.at[0]], rows_g.at[slot], gsem).wait()

    row_vecs = [iota + m * 16 for m in range(_BT // 16)]

    def transpose_chunk(gs, ps):
        def jloop(j, carry):
            jvec = jnp.full((16,), 0, jnp.int32) + j
            for m in range(_BT // 16):
                vec = plsc.load_gather(rows_g.at[gs], [row_vecs[m], jvec])
                plsc.store_scatter(pt.at[ps], [jvec, row_vecs[m]], vec)
            return carry
        lax.fori_loop(0, d, jloop, 0)

    def fire_stores(s_, ps):
        for jt in range(d // 8):
            pltpu.async_copy(
                pt.at[ps].at[pl.ds(8 * jt, 8), :],
                out_hbm.at[s_, jt, wid], ssem)

    def drain_stores():
        for _ in range(d // 8):
            pltpu.make_async_copy(
                pt.at[0].at[pl.ds(0, 8), :], out_hbm.at[0, 0, 0], ssem).wait()

    for b in range(_G):
        fire_gather(b, b)

    def outer(g, carry):
        for b in range(_NG):
            s_ = g * _NG + b

            @pl.when(s_ >= _NP)
            def _():
                drain_stores()

            @pl.when(s_ + _G < ns)
            def _():
                fire_gather(s_ + _G, (b + _G) % _NG)

            drain_gather(b)
            transpose_chunk(b, b % _NP)
            fire_stores(s_, b % _NP)
        return carry

    lax.fori_loop(0, ns // _NG, outer, 0)
    for _ in range(_NP):
        drain_stores()


def kernel(x, W):
    b, s = x.shape
    v, d = W.shape
    Wrm = jnp.ravel(W).reshape(v, d)
    mesh = plsc.VectorSubcoreMesh(core_axis_name="c", subcore_axis_name="s")
    out = pl.kernel(
        _emb_body,
        out_type=jax.ShapeDtypeStruct((s, d // 8, b // _BT, 8, _BT), jnp.float32),
        mesh=mesh,
        scratch_types=[
            pltpu.VMEM((_BT, s), jnp.int32),
            pltpu.VMEM((s, _BT), jnp.int32),
            pltpu.VMEM((_NG, _BT, d), jnp.float32),
            pltpu.VMEM((_NP, d, _BT), jnp.float32),
            pltpu.SemaphoreType.DMA,
            pltpu.SemaphoreType.DMA,
        ],
        compiler_params=pltpu.CompilerParams(use_tc_tiling_on_sc=False, needs_layout_passes=False),
    )(x, Wrm)
    return out.transpose(2, 4, 0, 1, 3).reshape(b, s, d)


# speedup vs baseline: 1.0002x; 1.0002x over previous
import jax
import jax.numpy as jnp
from jax import lax
from jax.experimental import pallas as pl
from jax.experimental.pallas import tpu as pltpu
from jax.experimental.pallas import tpu_sc as plsc

_NC, _NS = 2, 16
_NW = _NC * _NS
_BT = 128
_NG = 8
_NP = 2
_G = 6


def _emb_body(x_hbm, w_hbm, out_hbm, idx_v, idxt_v, rows_g, pt, gsem, ssem):
    wid = lax.axis_index("s") * _NC + lax.axis_index("c")
    ns = idxt_v.shape[0]
    d = w_hbm.shape[1]

    pltpu.sync_copy(x_hbm.at[pl.ds(wid * _BT, _BT)], idx_v)
    iota = jax.lax.iota(jnp.int32, 16)

    def sloop(s_, carry):
        scol = jnp.full((16,), 0, jnp.int32) + s_
        for m in range(_BT // 16):
            vec = plsc.load_gather(idx_v, [iota + m * 16, scol])
            plsc.store_scatter(idxt_v, [scol, iota + m * 16], vec)
        return carry

    lax.fori_loop(0, ns, sloop, 0)

    def fire_gather(s_, slot):
        pltpu.async_copy(w_hbm.at[idxt_v.at[s_]], rows_g.at[slot], gsem)

    def drain_gather(slot):
        pltpu.make_async_copy(
            w_hbm.at[idxt_v.at[0]], rows_g.at[slot], gsem).wait()

    row_vecs = [iota + m * 16 for m in range(_BT // 16)]

    def transpose_chunk(gs, ps):
        def jloop(j, carry):
            jvec = jnp.full((16,), 0, jnp.int32) + j
            for m in range(_BT // 16):
                vec = plsc.load_gather(rows_g.at[gs], [row_vecs[m], jvec])
                plsc.store_scatter(pt.at[ps], [jvec, row_vecs[m]], vec)
            return carry
        lax.fori_loop(0, d, jloop, 0)

    def fire_stores(s_, ps):
        for jt in range(d // 8):
            pltpu.async_copy(
                pt.at[ps].at[pl.ds(8 * jt, 8), :],
                out_hbm.at[s_, jt, wid], ssem)

    def drain_stores():
        for _ in range(d // 8):
            pltpu.make_async_copy(
                pt.at[0].at[pl.ds(0, 8), :], out_hbm.at[0, 0, 0], ssem).wait()

    for b in range(_G):
        fire_gather(b, b)

    def outer(g, carry):
        for b in range(_NG):
            s_ = g * _NG + b

            @pl.when(s_ >= _NP)
            def _():
                drain_stores()

            @pl.when(s_ + _G < ns)
            def _():
                fire_gather(s_ + _G, (b + _G) % _NG)

            drain_gather(b)
            transpose_chunk(b, b % _NP)
            fire_stores(s_, b % _NP)
        return carry

    lax.fori_loop(0, ns // _NG, outer, 0)
    for _ in range(_NP):
        drain_stores()


def kernel(x, W):
    b, s = x.shape
    v, d = W.shape
    Wrm = jnp.ravel(W).reshape(v, d)
    mesh = plsc.VectorSubcoreMesh(core_axis_name="c", subcore_axis_name="s")
    out = pl.kernel(
        _emb_body,
        out_type=jax.ShapeDtypeStruct((s, d // 8, b // _BT, 8, _BT), jnp.float32),
        mesh=mesh,
        scratch_types=[
            pltpu.VMEM((_BT, s), jnp.int32),
            pltpu.VMEM((s, _BT), jnp.int32),
            pltpu.VMEM((_NG, _BT, d), jnp.float32),
            pltpu.VMEM((_NP, d, _BT), jnp.float32),
            pltpu.SemaphoreType.DMA,
            pltpu.SemaphoreType.DMA,
        ],
        compiler_params=pltpu.CompilerParams(use_tc_tiling_on_sc=False, needs_layout_passes=False, disable_bounds_checks=True),
    )(x, Wrm)
    return out.transpose(2, 4, 0, 1, 3).reshape(b, s, d)


# speedup vs baseline: 1.1485x; 1.1483x over previous
import jax
import jax.numpy as jnp
from jax import lax
from jax.experimental import pallas as pl
from jax.experimental.pallas import tpu as pltpu
from jax.experimental.pallas import tpu_sc as plsc

_NC, _NS = 2, 16
_NW = _NC * _NS
_CH = 400
_NB = 4
_G = 2


def _emb_body(x_hbm, w_hbm, out_hbm, idx_v, rows_v, gsem, ssem):
    wid = lax.axis_index("s") * _NC + lax.axis_index("c")
    k = idx_v.shape[0]
    base = wid * k * _CH
    pltpu.sync_copy(x_hbm.at[pl.ds(wid * k, k)], idx_v)

    def fire_gather(j, slot):
        pltpu.async_copy(w_hbm.at[idx_v.at[j]], rows_v.at[slot], gsem)

    def fire_store(j, slot):
        pltpu.async_copy(rows_v.at[slot], out_hbm.at[pl.ds(base + j * _CH, _CH)], ssem)

    def drain_store():
        pltpu.make_async_copy(
            rows_v.at[0], out_hbm.at[pl.ds(base, _CH)], ssem).wait()

    def drain_gather(slot):
        pltpu.make_async_copy(
            w_hbm.at[idx_v.at[0]], rows_v.at[slot], gsem).wait()

    for b in range(_G):
        fire_gather(b, b)

    def outer(g, carry):
        for b in range(_NB):
            j = g * _NB + b

            @pl.when(j >= 1)
            def _():
                drain_store()

            @pl.when(j + _G < k)
            def _():
                fire_gather(j + _G, (b + _G) % _NB)

            drain_gather(b)
            fire_store(j, b)
        return carry

    lax.fori_loop(0, k // _NB, outer, 0)
    drain_store()


def kernel(x, W):
    b, s = x.shape
    v, d = W.shape
    n = b * s
    k = n // (_NW * _CH)
    x2 = x.reshape(_NW * k, _CH)
    mesh = plsc.VectorSubcoreMesh(core_axis_name="c", subcore_axis_name="s")
    out = pl.kernel(
        _emb_body,
        out_type=jax.ShapeDtypeStruct((n, d), jnp.float32),
        mesh=mesh,
        scratch_types=[
            pltpu.VMEM((k, _CH), jnp.int32),
            pltpu.VMEM((_NB, _CH, d), jnp.float32),
            pltpu.SemaphoreType.DMA,
            pltpu.SemaphoreType.DMA,
        ],
        compiler_params=pltpu.CompilerParams(use_tc_tiling_on_sc=False),
    )(x2, W)
    return out.reshape(b, s, d)


# speedup vs baseline: 1.3409x; 1.1675x over previous
import jax
import jax.numpy as jnp
from jax import lax
from jax.experimental import pallas as pl
from jax.experimental.pallas import tpu as pltpu
from jax.experimental.pallas import tpu_sc as plsc

_NC, _NS = 2, 16
_NW = _NC * _NS
_BT = 128
_NG = 8
_NP = 2
_G = 6
_DO_TRANSPOSE = True


def _emb_body(x_hbm, w_hbm, out_hbm, idx_v, idxt_v, rows_g, pt, gsem, ssem):
    wid = lax.axis_index("s") * _NC + lax.axis_index("c")
    ns = idxt_v.shape[0]
    d = w_hbm.shape[1]
    pltpu.sync_copy(x_hbm.at[pl.ds(wid * _BT, _BT)], idx_v)
    iota = jax.lax.iota(jnp.int32, 16)

    def sloop(s_, carry):
        scol = jnp.full((16,), 0, jnp.int32) + s_
        for m in range(_BT // 16):
            vec = plsc.load_gather(idx_v, [iota + m * 16, scol])
            plsc.store_scatter(idxt_v, [scol, iota + m * 16], vec)
        return carry

    lax.fori_loop(0, ns, sloop, 0)

    def fire_gather(s_, slot):
        pltpu.async_copy(w_hbm.at[idxt_v.at[s_]], rows_g.at[slot], gsem)

    def drain_gather(slot):
        pltpu.make_async_copy(
            w_hbm.at[idxt_v.at[0]], rows_g.at[slot], gsem).wait()

    row_vecs = [iota + m * 16 for m in range(_BT // 16)]

    def transpose_chunk(gs, ps):
        @plsc.parallel_loop(0, d, unroll=4)
        def _(j):
            jvec = jnp.full((16,), 0, jnp.int32) + j
            for m in range(_BT // 16):
                vec = plsc.load_gather(rows_g.at[gs], [row_vecs[m], jvec])
                plsc.store_scatter(pt.at[ps], [jvec, row_vecs[m]], vec)

    def fire_stores(s_, ps):
        for jt in range(d // 8):
            pltpu.async_copy(
                pt.at[ps].at[pl.ds(8 * jt, 8), :],
                out_hbm.at[s_, jt, wid], ssem)

    def drain_stores():
        for _ in range(d // 8):
            pltpu.make_async_copy(
                pt.at[0].at[pl.ds(0, 8), :], out_hbm.at[0, 0, 0], ssem).wait()

    for b in range(_G):
        fire_gather(b, b)

    def outer(g, carry):
        for b in range(_NG):
            s_ = g * _NG + b

            @pl.when(s_ >= _NP)
            def _():
                drain_stores()

            @pl.when(s_ + _G < ns)
            def _():
                fire_gather(s_ + _G, (b + _G) % _NG)

            drain_gather(b)
            if _DO_TRANSPOSE:
                transpose_chunk(b, b % _NP)
            fire_stores(s_, b % _NP)
        return carry

    lax.fori_loop(0, ns // _NG, outer, 0)
    for _ in range(_NP):
        drain_stores()


def kernel(x, W):
    b, s = x.shape
    v, d = W.shape
    Wrm = jnp.ravel(W).reshape(v, d)
    mesh = plsc.VectorSubcoreMesh(core_axis_name="c", subcore_axis_name="s")
    out = pl.kernel(
        _emb_body,
        out_type=jax.ShapeDtypeStruct((s, d // 8, b // _BT, 8, _BT), jnp.float32),
        mesh=mesh,
        scratch_types=[
            pltpu.VMEM((_BT, s), jnp.int32),
            pltpu.VMEM((s, _BT), jnp.int32),
            pltpu.VMEM((_NG, _BT, d), jnp.float32),
            pltpu.VMEM((_NP, d, _BT), jnp.float32),
            pltpu.SemaphoreType.DMA,
            pltpu.SemaphoreType.DMA,
        ],
        compiler_params=pltpu.CompilerParams(use_tc_tiling_on_sc=False, needs_layout_passes=False, disable_bounds_checks=True),
    )(x, Wrm)
    return out.transpose(2, 4, 0, 1, 3).reshape(b, s, d)
